# Initial kernel scaffold; baseline (speedup 1.0000x reference)
import jax
import jax.numpy as jnp
from jax.experimental import pallas as pl


def kernel(input, word_table):
    raise NotImplementedError("write your pallas kernel here")


# speedup vs baseline: 2.0213x; 2.0213x over previous
import functools
import math

import jax
import jax.numpy as jnp
import numpy as np
from jax import lax
from jax.experimental import pallas as pl
from jax.experimental.pallas import tpu as pltpu
from jax.experimental.pallas import tpu_sc as plsc

_VOCAB = 100000
_EMBED_DIM = 64
_SEQ = 200
_CHUNK = 128


def _make_sinusoidal_pe(max_len, embed_dim):
    pe = np.zeros((max_len, embed_dim), dtype=np.float32)
    position = np.arange(0, max_len, dtype=np.float32)[:, None]
    div_term = np.exp(
        np.arange(0, embed_dim, 2, dtype=np.float32) * -(math.log(10000.0) / embed_dim)
    )
    pe[:, 0::2] = np.sin(position * div_term)
    pe[:, 1::2] = np.cos(position * div_term)
    return pe


_PE_EXT = np.concatenate(
    [_make_sinusoidal_pe(_SEQ, _EMBED_DIM), _make_sinusoidal_pe(_SEQ, _EMBED_DIM)[:_CHUNK]],
    axis=0,
)


def _sc_embed(table, idx3d, pe_ext, *, n_rows):
    nc, ns = 2, 16
    nw = nc * ns
    assert idx3d.shape[0] == nw
    chunks_per_w = idx3d.shape[1]
    D = table.shape[1]

    mesh = plsc.VectorSubcoreMesh(
        core_axis_name="c", subcore_axis_name="s", num_cores=nc, num_subcores=ns
    )

    @functools.partial(
        pl.kernel,
        out_type=jax.ShapeDtypeStruct((n_rows, D), jnp.float32),
        mesh=mesh,
        scratch_types=[
            pltpu.VMEM((chunks_per_w, _CHUNK), jnp.int32),
            pltpu.VMEM((_CHUNK, D), jnp.float32),
            pltpu.VMEM((pe_ext.shape[0], D), jnp.float32),
            pltpu.SemaphoreType.DMA,
        ],
        compiler_params=pltpu.CompilerParams(use_tc_tiling_on_sc=False),
    )
    def k(table_hbm, idx_hbm, pe_hbm, out_hbm, idx_v, rows_v, pe_v, sem):
        wid = lax.axis_index("s") * nc + lax.axis_index("c")
        pltpu.sync_copy(pe_hbm, pe_v)
        pltpu.sync_copy(idx_hbm.at[wid], idx_v)

        def chunk_body(c, carry):
            base = (wid * chunks_per_w + c) * _CHUNK
            off = lax.rem(base, _SEQ)
            pltpu.async_copy(table_hbm.at[idx_v.at[c]], rows_v, sem).wait()

            def row_body(r, carry2):
                for j in range(D // 16):
                    sl = pl.ds(j * 16, 16)
                    rows_v[r, sl] = rows_v[r, sl] + pe_v[off + r, sl]
                return carry2

            lax.fori_loop(0, _CHUNK, row_body, 0, unroll=4)
            pltpu.sync_copy(rows_v, out_hbm.at[pl.ds(base, _CHUNK)])
            return carry

        lax.fori_loop(0, chunks_per_w, chunk_body, 0)

    return k(table, idx3d, pe_ext)


@jax.jit
def kernel(input, word_table):
    B, S = input.shape
    D = word_table.shape[1]
    n_rows = B * S
    nw = 32
    assert n_rows % (_CHUNK * nw) == 0
    idx3d = input.reshape(nw, n_rows // (_CHUNK * nw), _CHUNK)
    pe_ext = jnp.asarray(_PE_EXT[:, :D])
    out = _sc_embed(word_table, idx3d, pe_ext, n_rows=n_rows)
    return out.reshape(B, S, D)


# speedup vs baseline: 2.3513x; 1.1633x over previous
import functools
import math

import jax
import jax.numpy as jnp
import numpy as np
from jax import lax
from jax.experimental import pallas as pl
from jax.experimental.pallas import tpu as pltpu
from jax.experimental.pallas import tpu_sc as plsc

_VOCAB = 100000
_EMBED_DIM = 64
_SEQ = 200
_CHUNK = 128


def _make_sinusoidal_pe(max_len, embed_dim):
    pe = np.zeros((max_len, embed_dim), dtype=np.float32)
    position = np.arange(0, max_len, dtype=np.float32)[:, None]
    div_term = np.exp(
        np.arange(0, embed_dim, 2, dtype=np.float32) * -(math.log(10000.0) / embed_dim)
    )
    pe[:, 0::2] = np.sin(position * div_term)
    pe[:, 1::2] = np.cos(position * div_term)
    return pe


_PE_EXT = np.concatenate(
    [_make_sinusoidal_pe(_SEQ, _EMBED_DIM), _make_sinusoidal_pe(_SEQ, _EMBED_DIM)[:_CHUNK]],
    axis=0,
)


def _sc_embed(table, idx3d, pe_ext, *, n_rows):
    nc, ns = 2, 16
    nw = nc * ns
    assert idx3d.shape[0] == nw
    chunks_per_w = idx3d.shape[1]
    D = table.shape[1]

    mesh = plsc.VectorSubcoreMesh(
        core_axis_name="c", subcore_axis_name="s", num_cores=nc, num_subcores=ns
    )

    NBUF = 5
    LEAD = 2
    assert chunks_per_w % NBUF == 0 and chunks_per_w >= NBUF

    @functools.partial(
        pl.kernel,
        out_type=jax.ShapeDtypeStruct((n_rows, D), jnp.float32),
        mesh=mesh,
        scratch_types=[
            pltpu.VMEM((chunks_per_w, _CHUNK), jnp.int32),
            pltpu.VMEM((pe_ext.shape[0], D), jnp.float32),
        ]
        + [pltpu.VMEM((_CHUNK, D), jnp.float32) for _ in range(NBUF)]
        + [pltpu.SemaphoreType.DMA for _ in range(2 * NBUF)],
        compiler_params=pltpu.CompilerParams(use_tc_tiling_on_sc=False),
    )
    def k(table_hbm, idx_hbm, pe_hbm, out_hbm, idx_v, pe_v, *bufs_and_sems):
        rows = bufs_and_sems[:NBUF]
        gsem = bufs_and_sems[NBUF : 2 * NBUF]
        wsem = bufs_and_sems[2 * NBUF : 3 * NBUF]
        wid = lax.axis_index("s") * nc + lax.axis_index("c")
        base0 = wid * chunks_per_w
        pltpu.sync_copy(pe_hbm, pe_v)
        pltpu.sync_copy(idx_hbm.at[wid], idx_v)

        def gather(c, b):
            return pltpu.make_async_copy(table_hbm.at[idx_v.at[c]], rows[b], gsem[b])

        def writeback(c, b):
            return pltpu.make_async_copy(
                rows[b], out_hbm.at[pl.ds((base0 + c) * _CHUNK, _CHUNK)], wsem[b]
            )

        for b in range(LEAD):
            gather(b, b).start()

        def step(g, carry):
            for b in range(NBUF):
                c = g * NBUF + b
                gather(c, b).wait()
                off = lax.rem((base0 + c) * _CHUNK, _SEQ)

                def row_body(r, carry2):
                    for j in range(D // 16):
                        sl = pl.ds(j * 16, 16)
                        rows[b][r, sl] = rows[b][r, sl] + pe_v[off + r, sl]
                    return carry2

                lax.fori_loop(0, _CHUNK, row_body, 0, unroll=4)
                writeback(c, b).start()

                cf = c + LEAD
                bf = (b + LEAD) % NBUF

                @pl.when(cf < chunks_per_w)
                def _():
                    @pl.when(cf >= NBUF)
                    def _():
                        writeback(cf - NBUF, bf).wait()

                    gather(cf, bf).start()

            return carry

        lax.fori_loop(0, chunks_per_w // NBUF, step, 0)
        for b in range(NBUF):
            writeback(chunks_per_w - NBUF + b, b).wait()

    return k(table, idx3d, pe_ext)


@jax.jit
def kernel(input, word_table):
    B, S = input.shape
    D = word_table.shape[1]
    n_rows = B * S
    nw = 32
    assert n_rows % (_CHUNK * nw) == 0
    idx3d = input.reshape(nw, n_rows // (_CHUNK * nw), _CHUNK)
    pe_ext = jnp.asarray(_PE_EXT[:, :D])
    out = _sc_embed(word_table, idx3d, pe_ext, n_rows=n_rows)
    return out.reshape(B, S, D)


# speedup vs baseline: 2.4403x; 1.0379x over previous
import functools
import math

import jax
import jax.numpy as jnp
import numpy as np
from jax import lax
from jax.experimental import pallas as pl
from jax.experimental.pallas import tpu as pltpu
from jax.experimental.pallas import tpu_sc as plsc


def _make_sinusoidal_pe(max_len, embed_dim):
    pe = np.zeros((max_len, embed_dim), dtype=np.float32)
    position = np.arange(0, max_len, dtype=np.float32)[:, None]
    div_term = np.exp(
        np.arange(0, embed_dim, 2, dtype=np.float32) * -(math.log(10000.0) / embed_dim)
    )
    pe[:, 0::2] = np.sin(position * div_term)
    pe[:, 1::2] = np.cos(position * div_term)
    return pe


_PE = _make_sinusoidal_pe(512, 64)


def _sc_embed(table, idx, pe):
    nc, ns = 2, 16
    nw = nc * ns
    B, S = idx.shape
    D = table.shape[1]
    assert B % nw == 0
    seqs_per_w = B // nw
    assert S % 8 == 0
    pieces = []
    off = 0
    while off < S:
        sz = min(128, S - off)
        pieces.append((off, sz))
        off += sz

    mesh = plsc.VectorSubcoreMesh(
        core_axis_name="c", subcore_axis_name="s", num_cores=nc, num_subcores=ns
    )

    NBUF = 4
    LEAD = 2
    assert seqs_per_w % NBUF == 0 and seqs_per_w >= NBUF

    @functools.partial(
        pl.kernel,
        out_type=jax.ShapeDtypeStruct((B, S, D), jnp.float32),
        mesh=mesh,
        scratch_types=[
            pltpu.VMEM((seqs_per_w, S), jnp.int32),
            pltpu.VMEM((S, D), jnp.float32),
        ]
        + [pltpu.VMEM((S, D), jnp.float32) for _ in range(NBUF)]
        + [pltpu.SemaphoreType.DMA for _ in range(2 * NBUF)],
        compiler_params=pltpu.CompilerParams(use_tc_tiling_on_sc=False),
    )
    def k(table_hbm, idx_hbm, pe_hbm, out_hbm, idx_v, pe_v, *bufs_and_sems):
        rows = bufs_and_sems[:NBUF]
        gsem = bufs_and_sems[NBUF : 2 * NBUF]
        wsem = bufs_and_sems[2 * NBUF : 3 * NBUF]
        wid = lax.axis_index("s") * nc + lax.axis_index("c")
        seq0 = wid * seqs_per_w
        pltpu.sync_copy(pe_hbm, pe_v)
        pltpu.sync_copy(idx_hbm.at[pl.ds(seq0, seqs_per_w)], idx_v)

        def gather_pieces(s, b):
            return [
                pltpu.make_async_copy(
                    table_hbm.at[idx_v.at[s, pl.ds(p_off, p_sz)]],
                    rows[b].at[pl.ds(p_off, p_sz)],
                    gsem[b],
                )
                for p_off, p_sz in pieces
            ]

        def writeback(s, b):
            return pltpu.make_async_copy(rows[b], out_hbm.at[seq0 + s], wsem[b])

        def gather_start(s, b):
            for d in gather_pieces(s, b):
                d.start()

        def gather_wait(s, b):
            for d in gather_pieces(s, b):
                d.wait()

        for b in range(LEAD):
            gather_start(b, b)

        def step(g, carry):
            for b in range(NBUF):
                s = g * NBUF + b
                gather_wait(s, b)

                def row_body(r, carry2):
                    for j in range(D // 16):
                        sl = pl.ds(j * 16, 16)
                        rows[b][r, sl] = rows[b][r, sl] + pe_v[r, sl]
                    return carry2

                lax.fori_loop(0, S, row_body, 0, unroll=4)
                writeback(s, b).start()

                sf = s + LEAD
                bf = (b + LEAD) % NBUF

                @pl.when(sf < seqs_per_w)
                def _():
                    @pl.when(sf >= NBUF)
                    def _():
                        writeback(sf - NBUF, bf).wait()

                    gather_start(sf, bf)

            return carry

        lax.fori_loop(0, seqs_per_w // NBUF, step, 0)
        for b in range(NBUF):
            writeback(seqs_per_w - NBUF + b, b).wait()

    return k(table, idx, pe)


@jax.jit
def kernel(input, word_table):
    B, S = input.shape
    D = word_table.shape[1]
    pe = jnp.asarray(_PE[:S, :D])
    return _sc_embed(word_table, input, pe)


# speedup vs baseline: 2.8938x; 1.1858x over previous
import functools
import math

import jax
import jax.numpy as jnp
import numpy as np
from jax import lax
from jax.experimental import pallas as pl
from jax.experimental.pallas import tpu as pltpu
from jax.experimental.pallas import tpu_sc as plsc


def _make_sinusoidal_pe(max_len, embed_dim):
    pe = np.zeros((max_len, embed_dim), dtype=np.float32)
    position = np.arange(0, max_len, dtype=np.float32)[:, None]
    div_term = np.exp(
        np.arange(0, embed_dim, 2, dtype=np.float32) * -(math.log(10000.0) / embed_dim)
    )
    pe[:, 0::2] = np.sin(position * div_term)
    pe[:, 1::2] = np.cos(position * div_term)
    return pe


_PE = _make_sinusoidal_pe(512, 64)


def _sc_embed_planes(tableT, idxT, peT):
    nc, ns = 2, 16
    nw = nc * ns
    D, V = tableT.shape
    S, B = idxT.shape
    assert D % nw == 0 and B == 1024 and S % 2 == 0
    passes = D // nw

    mesh = plsc.VectorSubcoreMesh(
        core_axis_name="c", subcore_axis_name="s", num_cores=nc, num_subcores=ns
    )

    @functools.partial(
        pl.kernel,
        out_type=jax.ShapeDtypeStruct((S, 8, 8, 8, 128), jnp.float32),
        mesh=mesh,
        scratch_types=[
            pltpu.VMEM((S,), jnp.float32),
            pltpu.VMEM((V,), jnp.float32),
            pltpu.VMEM((B,), jnp.int32),
            pltpu.VMEM((B,), jnp.int32),
            pltpu.VMEM((8, 1, 128), jnp.float32),
            pltpu.VMEM((8, 1, 128), jnp.float32),
            pltpu.SemaphoreType.DMA,
            pltpu.SemaphoreType.DMA,
            pltpu.SemaphoreType.DMA,
            pltpu.SemaphoreType.DMA,
        ],
        compiler_params=pltpu.CompilerParams(
            use_tc_tiling_on_sc=False, needs_layout_passes=False
        ),
    )
    def k(tableT_hbm, idxT_hbm, peT_hbm, out_hbm, pe_s, trow_v, i0, i1, o0, o1,
          is0, is1, os0, os1):
        ibuf = (i0, i1)
        obuf = (o0, o1)
        isem = (is0, is1)
        osem = (os0, os1)
        wid = lax.axis_index("s") * nc + lax.axis_index("c")

        def idx_fetch(s, b):
            return pltpu.make_async_copy(idxT_hbm.at[s], ibuf[b], isem[b])

        for p in range(passes):
            d = wid + p * nw
            dh = d // 8
            dl = lax.rem(d, 8)
            pltpu.sync_copy(peT_hbm.at[d], pe_s)
            pltpu.sync_copy(tableT_hbm.at[d], trow_v)

            def writeback(s, b, dh=dh, dl=dl):
                return pltpu.make_async_copy(
                    obuf[b],
                    out_hbm.at[s, dh, :, pl.ds(dl, 1), :],
                    osem[b],
                )

            for b in range(2):
                idx_fetch(b, b).start()

            def step(g, carry, d=d, writeback=writeback):
                for b in range(2):
                    s = 2 * g + b
                    idx_fetch(s, b).wait()

                    @pl.when(s >= 2)
                    def _():
                        writeback(s - 2, b).wait()

                    vpe = plsc.load_gather(pe_s, [jnp.full((16,), s, jnp.int32)])

                    def row_body(r, carry2):
                        for jj in range(8):
                            idxv = ibuf[b][pl.ds(r * 128 + jj * 16, 16)]
                            vals = plsc.load_gather(trow_v, [idxv])
                            obuf[b][r, 0, pl.ds(jj * 16, 16)] = vals + vpe
                        return carry2

                    lax.fori_loop(0, 8, row_body, 0)
                    writeback(s, b).start()

                    @pl.when(s + 2 < S)
                    def _():
                        idx_fetch(s + 2, b).start()

                return carry

            lax.fori_loop(0, S // 2, step, 0)
            for b in range(2):
                writeback(S - 2 + b, b).wait()

    return k(tableT, idxT, peT)


@jax.jit
def kernel(input, word_table):
    B, S = input.shape
    D = word_table.shape[1]
    peT = jnp.asarray(np.ascontiguousarray(_PE[:S, :D].T))
    o5 = _sc_embed_planes(word_table.T, input.T, peT)
    return o5.transpose(2, 4, 0, 1, 3).reshape(B, S, D)


# speedup vs baseline: 2.9044x; 1.0037x over previous
import functools
import math

import jax
import jax.numpy as jnp
import numpy as np
from jax import lax
from jax.experimental import pallas as pl
from jax.experimental.pallas import tpu as pltpu
from jax.experimental.pallas import tpu_sc as plsc


def _make_sinusoidal_pe(max_len, embed_dim):
    pe = np.zeros((max_len, embed_dim), dtype=np.float32)
    position = np.arange(0, max_len, dtype=np.float32)[:, None]
    div_term = np.exp(
        np.arange(0, embed_dim, 2, dtype=np.float32) * -(math.log(10000.0) / embed_dim)
    )
    pe[:, 0::2] = np.sin(position * div_term)
    pe[:, 1::2] = np.cos(position * div_term)
    return pe


_PE = _make_sinusoidal_pe(512, 64)


def _sc_embed_planes(tableT, idxT, peT):
    nc, ns = 2, 16
    nw = nc * ns
    D, V = tableT.shape
    S, B = idxT.shape
    assert D % nw == 0 and B == 1024 and S % 2 == 0
    passes = D // nw

    mesh = plsc.VectorSubcoreMesh(
        core_axis_name="c", subcore_axis_name="s", num_cores=nc, num_subcores=ns
    )

    @functools.partial(
        pl.kernel,
        out_type=jax.ShapeDtypeStruct((S, 8, 8, 8, 128), jnp.float32),
        mesh=mesh,
        scratch_types=[
            pltpu.VMEM((S,), jnp.float32),
            pltpu.VMEM((V,), jnp.float32),
            pltpu.VMEM((B,), jnp.int32),
            pltpu.VMEM((B,), jnp.int32),
            pltpu.VMEM((8, 1, 128), jnp.float32),
            pltpu.VMEM((8, 1, 128), jnp.float32),
            pltpu.SemaphoreType.DMA,
            pltpu.SemaphoreType.DMA,
            pltpu.SemaphoreType.DMA,
            pltpu.SemaphoreType.DMA,
        ],
        compiler_params=pltpu.CompilerParams(
            use_tc_tiling_on_sc=False, needs_layout_passes=False
        ),
    )
    def k(tableT_hbm, idxT_hbm, peT_hbm, out_hbm, pe_s, trow_v, i0, i1, o0, o1,
          is0, is1, os0, os1):
        ibuf = (i0, i1)
        obuf = (o0, o1)
        isem = (is0, is1)
        osem = (os0, os1)
        wid = lax.axis_index("s") * nc + lax.axis_index("c")

        def idx_fetch(s, b):
            return pltpu.make_async_copy(idxT_hbm.at[s], ibuf[b], isem[b])

        for p in range(passes):
            d = wid + p * nw
            dh = d // 8
            dl = lax.rem(d, 8)
            pltpu.sync_copy(peT_hbm.at[d], pe_s)
            pltpu.sync_copy(tableT_hbm.at[d], trow_v)

            def writeback(s, b, dh=dh, dl=dl):
                return pltpu.make_async_copy(
                    obuf[b],
                    out_hbm.at[s, dh, :, pl.ds(dl, 1), :],
                    osem[b],
                )

            for b in range(2):
                idx_fetch(b, b).start()

            def step(g, carry, d=d, writeback=writeback):
                for b in range(2):
                    s = 2 * g + b
                    idx_fetch(s, b).wait()

                    @pl.when(s >= 2)
                    def _():
                        writeback(s - 2, b).wait()

                    vpe = plsc.load_gather(pe_s, [jnp.full((16,), s, jnp.int32)])

                    for r in range(8):
                        for jj in range(8):
                            idxv = ibuf[b][pl.ds(r * 128 + jj * 16, 16)]
                            vals = plsc.load_gather(trow_v, [idxv])
                            obuf[b][r, 0, pl.ds(jj * 16, 16)] = vals + vpe
                    writeback(s, b).start()

                    @pl.when(s + 2 < S)
                    def _():
                        idx_fetch(s + 2, b).start()

                return carry

            lax.fori_loop(0, S // 2, step, 0)
            for b in range(2):
                writeback(S - 2 + b, b).wait()

    return k(tableT, idxT, peT)


@jax.jit
def kernel(input, word_table):
    B, S = input.shape
    D = word_table.shape[1]
    peT = jnp.asarray(np.ascontiguousarray(_PE[:S, :D].T))
    o5 = _sc_embed_planes(word_table.T, input.T, peT)
    return o5.transpose(2, 4, 0, 1, 3).reshape(B, S, D)


# speedup vs baseline: 3.2996x; 1.1361x over previous
import functools
import math

import jax
import jax.numpy as jnp
import numpy as np
from jax import lax
from jax.experimental import pallas as pl
from jax.experimental.pallas import tpu as pltpu
from jax.experimental.pallas import tpu_sc as plsc


def _make_sinusoidal_pe(max_len, embed_dim):
    pe = np.zeros((max_len, embed_dim), dtype=np.float32)
    position = np.arange(0, max_len, dtype=np.float32)[:, None]
    div_term = np.exp(
        np.arange(0, embed_dim, 2, dtype=np.float32) * -(math.log(10000.0) / embed_dim)
    )
    pe[:, 0::2] = np.sin(position * div_term)
    pe[:, 1::2] = np.cos(position * div_term)
    return pe


_PE = _make_sinusoidal_pe(512, 64)

_RING = 8


def _sc_embed_planes(tableT, idxT, peT):
    nc, ns = 2, 16
    nw = nc * ns
    D, V = tableT.shape
    S, B = idxT.shape
    R = _RING
    assert D % nw == 0 and B == 1024 and S % R == 0
    passes = D // nw

    mesh = plsc.VectorSubcoreMesh(
        core_axis_name="c", subcore_axis_name="s", num_cores=nc, num_subcores=ns
    )

    @functools.partial(
        pl.kernel,
        out_type=jax.ShapeDtypeStruct((S, 8, 8, 8, 128), jnp.float32),
        mesh=mesh,
        scratch_types=[
            pltpu.VMEM((S,), jnp.float32),
            pltpu.VMEM((V,), jnp.float32),
            pltpu.VMEM((R, B), jnp.int32),
            pltpu.VMEM((R, 8, 1, 128), jnp.float32),
        ]
        + [pltpu.SemaphoreType.DMA for _ in range(2 * R)],
        compiler_params=pltpu.CompilerParams(
            use_tc_tiling_on_sc=False, needs_layout_passes=False
        ),
    )
    def k(tableT_hbm, idxT_hbm, peT_hbm, out_hbm, pe_s, trow_v, ibuf, obuf, *sems):
        isem = sems[:R]
        osem = sems[R : 2 * R]
        wid = lax.axis_index("s") * nc + lax.axis_index("c")

        def idx_fetch(s, q):
            return pltpu.make_async_copy(idxT_hbm.at[s], ibuf.at[q], isem[q])

        for p in range(passes):
            d = wid + p * nw
            dh = d // 8
            dl = lax.rem(d, 8)
            pltpu.sync_copy(peT_hbm.at[d], pe_s)
            pltpu.sync_copy(tableT_hbm.at[d], trow_v)

            def writeback(s, q, dh=dh, dl=dl):
                return pltpu.make_async_copy(
                    obuf.at[q],
                    out_hbm.at[s, dh, :, pl.ds(dl, 1), :],
                    osem[q],
                )

            for q in range(R):
                idx_fetch(q, q).start()

            def step(g, carry, d=d, writeback=writeback):
                for q in range(R):
                    s = R * g + q
                    idx_fetch(s, q).wait()

                    @pl.when(s >= R)
                    def _():
                        writeback(s - R, q).wait()

                    vpe = plsc.load_gather(pe_s, [jnp.full((16,), s, jnp.int32)])

                    for r in range(8):
                        for jj in range(8):
                            idxv = ibuf[q, pl.ds(r * 128 + jj * 16, 16)]
                            vals = plsc.load_gather(trow_v, [idxv])
                            obuf[q, r, 0, pl.ds(jj * 16, 16)] = vals + vpe
                    writeback(s, q).start()

                    @pl.when(s + R < S)
                    def _():
                        idx_fetch(s + R, q).start()

                return carry

            lax.fori_loop(0, S // R, step, 0)
            for q in range(R):
                writeback(S - R + q, q).wait()

    return k(tableT, idxT, peT)


@jax.jit
def kernel(input, word_table):
    B, S = input.shape
    D = word_table.shape[1]
    peT = jnp.asarray(np.ascontiguousarray(_PE[:S, :D].T))
    o5 = _sc_embed_planes(word_table.T, input.T, peT)
    return o5.transpose(2, 4, 0, 1, 3).reshape(B, S, D)
